# baseline (speedup 1.0000x reference)
import functools
import jax
import jax.numpy as jnp
from jax import lax
from jax.experimental import pallas as pl
from jax.experimental.pallas import tpu as pltpu
from jax.experimental.pallas import tpu_sc as plsc

N = 10000
E = 320000
D = 128
NC = 2
NS = 16
NW = NC * NS
EPW = E // NW
C = 100
NCH = EPW // C
NB = 3
M = 6
UWB = 624


def _seg_sum_body(eidx_hbm, table_hbm, zeros_hbm, out_hbm, *scr):
    acc_sh = scr[0]
    idx = scr[1:1 + M]
    rows = scr[1 + M:1 + M + NB]
    semz = scr[1 + M + NB]
    semi = scr[2 + M + NB:2 + 2 * M + NB]
    semg = scr[2 + 2 * M + NB:2 + 2 * M + 2 * NB]
    sems = scr[2 + 2 * M + 2 * NB:2 + 2 * M + 3 * NB]

    cid = lax.axis_index("c")
    sid = lax.axis_index("s")
    wid = cid * NS + sid
    row0 = sid * UWB

    @pl.when(sid < NS - 1)
    def _zero():
        pltpu.async_copy(zeros_hbm.at[pl.ds(0, UWB)],
                         acc_sh.at[pl.ds(row0, UWB)], semz)

    @pl.when(sid == NS - 1)
    def _zero_tail():
        pltpu.async_copy(zeros_hbm, acc_sh.at[pl.ds(row0, UWB + 16)], semz)

    for k in range(NB):
        pltpu.async_copy(eidx_hbm.at[wid, k], idx[k], semi[k])

    def _wait_idx(ch_static, b):
        pltpu.make_async_copy(eidx_hbm.at[wid, ch_static], idx[b],
                              semi[b]).wait()

    @pl.when(sid < NS - 1)
    def _zero_wait():
        pltpu.make_async_copy(zeros_hbm.at[pl.ds(0, UWB)],
                              acc_sh.at[pl.ds(row0, UWB)], semz).wait()

    @pl.when(sid == NS - 1)
    def _zero_tail_wait():
        pltpu.make_async_copy(zeros_hbm,
                              acc_sh.at[pl.ds(row0, UWB + 16)], semz).wait()

    for b in range(NB - 1):
        _wait_idx(b, b)
        pltpu.async_copy(acc_sh.at[idx[b].at[0]], rows[b], semg[b])

    plsc.subcore_barrier()

    def _chunk(ch, b, i8):
        pltpu.make_async_copy(acc_sh.at[idx[i8].at[0]], rows[b],
                              semg[b]).wait()
        pltpu.async_copy(rows[b], acc_sh.at[idx[i8].at[1]], sems[b],
                         add=True)

        @pl.when(ch + NB < NCH)
        def _():
            i_n = (i8 + NB) % M
            pltpu.async_copy(eidx_hbm.at[wid, ch + NB], idx[i_n], semi[i_n])

        @pl.when(ch + NB - 1 < NCH)
        def _():
            b2 = (b + NB - 1) % NB
            @pl.when(ch >= 1)
            def _():
                pltpu.make_async_copy(rows[b2], acc_sh.at[idx[0].at[1]],
                                      sems[b2]).wait()
            i2 = (i8 + NB - 1) % M
            pltpu.make_async_copy(eidx_hbm.at[wid, 0], idx[i2],
                                  semi[i2]).wait()
            pltpu.async_copy(acc_sh.at[idx[i2].at[0]], rows[b2], semg[b2])

    @pl.loop(0, NCH // M)
    def _edges(g):
        ch0 = M * g
        for j in range(M):
            _chunk(ch0 + j, j % NB, j)

    for ch in range(NCH - NCH % M, NCH):
        _chunk(jnp.int32(ch), ch % NB, ch % M)

    for b in range(NB):
        pltpu.make_async_copy(rows[b], acc_sh.at[idx[0].at[1]],
                              sems[b]).wait()

    plsc.subcore_barrier()

    @pl.when(sid < NS - 1)
    def _wb():
        pltpu.sync_copy(acc_sh.at[pl.ds(row0, UWB)],
                        out_hbm.at[cid, pl.ds(row0, UWB)])

    @pl.when(sid == NS - 1)
    def _wb_tail():
        pltpu.sync_copy(acc_sh.at[pl.ds(row0, UWB + 16)],
                        out_hbm.at[cid, pl.ds(row0, UWB + 16)])


@functools.cache
def _make_seg_sum():
  return pl.kernel(
    _seg_sum_body,
    out_type=jax.ShapeDtypeStruct((NC, N, D), jnp.float32),
    mesh=plsc.VectorSubcoreMesh(core_axis_name="c", subcore_axis_name="s",
                                num_cores=NC, num_subcores=NS),
    scratch_types=(
        [pltpu.VMEM_SHARED((N, D), jnp.float32)]
        + [pltpu.VMEM((2, C), jnp.int32) for _ in range(M)]
        + [pltpu.VMEM((C, D), jnp.float32) for _ in range(NB)]
        + [pltpu.SemaphoreType.DMA] * (1 + M + NB + NB)
    ),
  )


BLK = 1000
GRID = N // BLK


def _layer1_body(p0, p1, x, w_rel, b_rel, w_root, out):
    agg = p0[...] + p1[...]
    h = (jnp.dot(agg, w_rel[...], preferred_element_type=jnp.float32)
         + jnp.dot(x[...], w_root[...], preferred_element_type=jnp.float32)
         + b_rel[...])
    out[...] = jnp.maximum(h, 0.0)


_layer1 = pl.pallas_call(
    _layer1_body,
    grid=(GRID,),
    in_specs=[
        pl.BlockSpec((BLK, D), lambda i: (i, 0)),
        pl.BlockSpec((BLK, D), lambda i: (i, 0)),
        pl.BlockSpec((BLK, D), lambda i: (i, 0)),
        pl.BlockSpec((D, D), lambda i: (0, 0)),
        pl.BlockSpec((1, D), lambda i: (0, 0)),
        pl.BlockSpec((D, D), lambda i: (0, 0)),
    ],
    out_specs=pl.BlockSpec((BLK, D), lambda i: (i, 0)),
    out_shape=jax.ShapeDtypeStruct((N, D), jnp.float32),
)


def _layer2_body(p0, p1, h1, w_rel, b_rel, w_root, w_count, b_count,
                 w_jet, b_jet, cnt_out, props_out, acc):
    i = pl.program_id(0)
    agg = p0[...] + p1[...]
    h = (jnp.dot(agg, w_rel[...], preferred_element_type=jnp.float32)
         + jnp.dot(h1[...], w_root[...], preferred_element_type=jnp.float32)
         + b_rel[...])
    h = jnp.maximum(h, 0.0)
    blk_sum = jnp.sum(h, axis=0, keepdims=True)

    @pl.when(i == 0)
    def _():
        acc[...] = jnp.zeros_like(acc)

    acc[...] += blk_sum

    @pl.when(i == GRID - 1)
    def _():
        pooled = acc[...] / float(N)
        cnt = jnp.dot(pooled, w_count[...],
                      preferred_element_type=jnp.float32) + b_count[...]
        cnt_out[...] = jnp.round(jnp.clip(cnt, 0.0, 20.0))
        props = jnp.dot(pooled, w_jet[...],
                        preferred_element_type=jnp.float32) + b_jet[...]
        props_out[...] = jnp.broadcast_to(props, (3, 5))


_layer2 = pl.pallas_call(
    _layer2_body,
    grid=(GRID,),
    in_specs=[
        pl.BlockSpec((BLK, D), lambda i: (i, 0)),
        pl.BlockSpec((BLK, D), lambda i: (i, 0)),
        pl.BlockSpec((BLK, D), lambda i: (i, 0)),
        pl.BlockSpec((D, D), lambda i: (0, 0)),
        pl.BlockSpec((1, D), lambda i: (0, 0)),
        pl.BlockSpec((D, D), lambda i: (0, 0)),
        pl.BlockSpec((D, 1), lambda i: (0, 0)),
        pl.BlockSpec((1, 1), lambda i: (0, 0)),
        pl.BlockSpec((D, 5), lambda i: (0, 0)),
        pl.BlockSpec((1, 5), lambda i: (0, 0)),
    ],
    out_specs=[
        pl.BlockSpec((1, 1), lambda i: (0, 0)),
        pl.BlockSpec((3, 5), lambda i: (0, 0)),
    ],
    out_shape=[
        jax.ShapeDtypeStruct((1, 1), jnp.float32),
        jax.ShapeDtypeStruct((3, 5), jnp.float32),
    ],
    scratch_shapes=[pltpu.VMEM((1, D), jnp.float32)],
)


@jax.jit
def kernel(x, edge_index, batch, W1_rel, b1_rel, W1_root, W2_rel, b2_rel,
           W2_root, W_count, b_count, W_jet, b_jet):
    eidx = jnp.transpose(edge_index.reshape(2, NW, NCH, C), (1, 2, 0, 3))
    zeros = jnp.zeros((UWB + 16, D), jnp.float32)

    _seg_sum = _make_seg_sum()
    parts1 = _seg_sum(eidx, x, zeros)
    h1 = _layer1(parts1[0], parts1[1], x, W1_rel, b1_rel.reshape(1, D),
                 W1_root)
    parts2 = _seg_sum(eidx, h1, zeros)
    jet_count, jet_properties = _layer2(
        parts2[0], parts2[1], h1, W2_rel, b2_rel.reshape(1, D), W2_root,
        W_count, b_count.reshape(1, 1), W_jet, b_jet.reshape(1, 5))
    return (jet_count, jet_properties)


# speedup vs baseline: 1.1128x; 1.1128x over previous
import functools
import jax
import jax.numpy as jnp
from jax import lax
from jax.experimental import pallas as pl
from jax.experimental.pallas import tpu as pltpu
from jax.experimental.pallas import tpu_sc as plsc

N = 10000
E = 320000
D = 128
NC = 2
NS = 16
NW = NC * NS
EPW = E // NW
C = 100
NCH = EPW // C
NB = 3
M = 6
UWB = 624


def _seg_sum_body(eidx_hbm, table_hbm, zeros_hbm, out_hbm, *scr):
    acc_sh = scr[0]
    idx = scr[1:1 + M]
    rows = scr[1 + M:1 + M + NB]
    semz = scr[1 + M + NB]
    semi = scr[2 + M + NB:2 + 2 * M + NB]
    semg = scr[2 + 2 * M + NB:2 + 2 * M + 2 * NB]
    sems = scr[2 + 2 * M + 2 * NB:2 + 2 * M + 3 * NB]

    cid = lax.axis_index("c")
    sid = lax.axis_index("s")
    wid = cid * NS + sid
    row0 = sid * UWB

    @pl.when(sid < NS - 1)
    def _zero():
        pltpu.async_copy(zeros_hbm.at[pl.ds(0, UWB)],
                         acc_sh.at[pl.ds(row0, UWB)], semz)

    @pl.when(sid == NS - 1)
    def _zero_tail():
        pltpu.async_copy(zeros_hbm, acc_sh.at[pl.ds(row0, UWB + 16)], semz)

    for k in range(NB):
        pltpu.async_copy(eidx_hbm.at[wid, k], idx[k], semi[k])

    def _wait_idx(ch_static, b):
        pltpu.make_async_copy(eidx_hbm.at[wid, ch_static], idx[b],
                              semi[b]).wait()

    @pl.when(sid < NS - 1)
    def _zero_wait():
        pltpu.make_async_copy(zeros_hbm.at[pl.ds(0, UWB)],
                              acc_sh.at[pl.ds(row0, UWB)], semz).wait()

    @pl.when(sid == NS - 1)
    def _zero_tail_wait():
        pltpu.make_async_copy(zeros_hbm,
                              acc_sh.at[pl.ds(row0, UWB + 16)], semz).wait()

    for b in range(NB - 1):
        _wait_idx(b, b)
        pltpu.async_copy(table_hbm.at[idx[b].at[0]], rows[b], semg[b])

    plsc.subcore_barrier()

    def _chunk(ch, b, i8):
        pltpu.make_async_copy(table_hbm.at[idx[i8].at[0]], rows[b],
                              semg[b]).wait()
        pltpu.async_copy(rows[b], acc_sh.at[idx[i8].at[1]], sems[b],
                         add=True)

        @pl.when(ch + NB < NCH)
        def _():
            i_n = (i8 + NB) % M
            pltpu.async_copy(eidx_hbm.at[wid, ch + NB], idx[i_n], semi[i_n])

        @pl.when(ch + NB - 1 < NCH)
        def _():
            b2 = (b + NB - 1) % NB
            @pl.when(ch >= 1)
            def _():
                pltpu.make_async_copy(rows[b2], acc_sh.at[idx[0].at[1]],
                                      sems[b2]).wait()
            i2 = (i8 + NB - 1) % M
            pltpu.make_async_copy(eidx_hbm.at[wid, 0], idx[i2],
                                  semi[i2]).wait()
            pltpu.async_copy(table_hbm.at[idx[i2].at[0]], rows[b2], semg[b2])

    @pl.loop(0, NCH // M)
    def _edges(g):
        ch0 = M * g
        for j in range(M):
            _chunk(ch0 + j, j % NB, j)

    for ch in range(NCH - NCH % M, NCH):
        _chunk(jnp.int32(ch), ch % NB, ch % M)

    for b in range(NB):
        pltpu.make_async_copy(rows[b], acc_sh.at[idx[0].at[1]],
                              sems[b]).wait()

    plsc.subcore_barrier()

    @pl.when(sid < NS - 1)
    def _wb():
        pltpu.sync_copy(acc_sh.at[pl.ds(row0, UWB)],
                        out_hbm.at[cid, pl.ds(row0, UWB)])

    @pl.when(sid == NS - 1)
    def _wb_tail():
        pltpu.sync_copy(acc_sh.at[pl.ds(row0, UWB + 16)],
                        out_hbm.at[cid, pl.ds(row0, UWB + 16)])


@functools.cache
def _make_seg_sum():
  return pl.kernel(
    _seg_sum_body,
    out_type=jax.ShapeDtypeStruct((NC, N, D), jnp.float32),
    mesh=plsc.VectorSubcoreMesh(core_axis_name="c", subcore_axis_name="s",
                                num_cores=NC, num_subcores=NS),
    scratch_types=(
        [pltpu.VMEM_SHARED((N, D), jnp.float32)]
        + [pltpu.VMEM((2, C), jnp.int32) for _ in range(M)]
        + [pltpu.VMEM((C, D), jnp.float32) for _ in range(NB)]
        + [pltpu.SemaphoreType.DMA] * (1 + M + NB + NB)
    ),
  )


BLK = 1000
GRID = N // BLK


def _layer1_body(p0, p1, x, w_rel, b_rel, w_root, out):
    agg = p0[...] + p1[...]
    h = (jnp.dot(agg, w_rel[...], preferred_element_type=jnp.float32)
         + jnp.dot(x[...], w_root[...], preferred_element_type=jnp.float32)
         + b_rel[...])
    out[...] = jnp.maximum(h, 0.0)


_layer1 = pl.pallas_call(
    _layer1_body,
    grid=(GRID,),
    in_specs=[
        pl.BlockSpec((BLK, D), lambda i: (i, 0)),
        pl.BlockSpec((BLK, D), lambda i: (i, 0)),
        pl.BlockSpec((BLK, D), lambda i: (i, 0)),
        pl.BlockSpec((D, D), lambda i: (0, 0)),
        pl.BlockSpec((1, D), lambda i: (0, 0)),
        pl.BlockSpec((D, D), lambda i: (0, 0)),
    ],
    out_specs=pl.BlockSpec((BLK, D), lambda i: (i, 0)),
    out_shape=jax.ShapeDtypeStruct((N, D), jnp.float32),
)


def _layer2_body(p0, p1, h1, w_rel, b_rel, w_root, w_count, b_count,
                 w_jet, b_jet, cnt_out, props_out, acc):
    i = pl.program_id(0)
    agg = p0[...] + p1[...]
    h = (jnp.dot(agg, w_rel[...], preferred_element_type=jnp.float32)
         + jnp.dot(h1[...], w_root[...], preferred_element_type=jnp.float32)
         + b_rel[...])
    h = jnp.maximum(h, 0.0)
    blk_sum = jnp.sum(h, axis=0, keepdims=True)

    @pl.when(i == 0)
    def _():
        acc[...] = jnp.zeros_like(acc)

    acc[...] += blk_sum

    @pl.when(i == GRID - 1)
    def _():
        pooled = acc[...] / float(N)
        cnt = jnp.dot(pooled, w_count[...],
                      preferred_element_type=jnp.float32) + b_count[...]
        cnt_out[...] = jnp.round(jnp.clip(cnt, 0.0, 20.0))
        props = jnp.dot(pooled, w_jet[...],
                        preferred_element_type=jnp.float32) + b_jet[...]
        props_out[...] = jnp.broadcast_to(props, (3, 5))


_layer2 = pl.pallas_call(
    _layer2_body,
    grid=(GRID,),
    in_specs=[
        pl.BlockSpec((BLK, D), lambda i: (i, 0)),
        pl.BlockSpec((BLK, D), lambda i: (i, 0)),
        pl.BlockSpec((BLK, D), lambda i: (i, 0)),
        pl.BlockSpec((D, D), lambda i: (0, 0)),
        pl.BlockSpec((1, D), lambda i: (0, 0)),
        pl.BlockSpec((D, D), lambda i: (0, 0)),
        pl.BlockSpec((D, 1), lambda i: (0, 0)),
        pl.BlockSpec((1, 1), lambda i: (0, 0)),
        pl.BlockSpec((D, 5), lambda i: (0, 0)),
        pl.BlockSpec((1, 5), lambda i: (0, 0)),
    ],
    out_specs=[
        pl.BlockSpec((1, 1), lambda i: (0, 0)),
        pl.BlockSpec((3, 5), lambda i: (0, 0)),
    ],
    out_shape=[
        jax.ShapeDtypeStruct((1, 1), jnp.float32),
        jax.ShapeDtypeStruct((3, 5), jnp.float32),
    ],
    scratch_shapes=[pltpu.VMEM((1, D), jnp.float32)],
)


@jax.jit
def kernel(x, edge_index, batch, W1_rel, b1_rel, W1_root, W2_rel, b2_rel,
           W2_root, W_count, b_count, W_jet, b_jet):
    eidx = jnp.transpose(edge_index.reshape(2, NW, NCH, C), (1, 2, 0, 3))
    zeros = jnp.zeros((UWB + 16, D), jnp.float32)

    _seg_sum = _make_seg_sum()
    parts1 = _seg_sum(eidx, x, zeros)
    h1 = _layer1(parts1[0], parts1[1], x, W1_rel, b1_rel.reshape(1, D),
                 W1_root)
    parts2 = _seg_sum(eidx, h1, zeros)
    jet_count, jet_properties = _layer2(
        parts2[0], parts2[1], h1, W2_rel, b2_rel.reshape(1, D), W2_root,
        W_count, b_count.reshape(1, 1), W_jet, b_jet.reshape(1, 5))
    return (jet_count, jet_properties)


# speedup vs baseline: 1.1208x; 1.0072x over previous
import functools
import jax
import jax.numpy as jnp
from jax import lax
from jax.experimental import pallas as pl
from jax.experimental.pallas import tpu as pltpu
from jax.experimental.pallas import tpu_sc as plsc

N = 10000
E = 320000
D = 128
NC = 2
NS = 16
NW = NC * NS
EPW = E // NW
C = 100
NCH = EPW // C
NB = 3
M = 6
UWB = 624


def _seg_sum_body(eidx_hbm, table_hbm, zeros_hbm, out_hbm, *scr):
    acc_sh = scr[0]
    idx = scr[1:1 + M]
    rows = scr[1 + M:1 + M + NB]
    semz = scr[1 + M + NB]
    semi = scr[2 + M + NB:2 + 2 * M + NB]
    semg = scr[2 + 2 * M + NB:2 + 2 * M + 2 * NB]
    sems = scr[2 + 2 * M + 2 * NB:2 + 2 * M + 3 * NB]

    cid = lax.axis_index("c")
    sid = lax.axis_index("s")
    wid = cid * NS + sid
    row0 = sid * UWB

    @pl.when(sid < NS - 1)
    def _zero():
        pltpu.async_copy(zeros_hbm.at[pl.ds(0, UWB)],
                         acc_sh.at[pl.ds(row0, UWB)], semz)

    @pl.when(sid == NS - 1)
    def _zero_tail():
        pltpu.async_copy(zeros_hbm, acc_sh.at[pl.ds(row0, UWB + 16)], semz)

    for k in range(NB):
        pltpu.async_copy(eidx_hbm.at[pl.ds(0, 2), wid, k], idx[k], semi[k])

    def _wait_idx(ch_static, b):
        pltpu.make_async_copy(eidx_hbm.at[pl.ds(0, 2), wid, ch_static], idx[b],
                              semi[b]).wait()

    @pl.when(sid < NS - 1)
    def _zero_wait():
        pltpu.make_async_copy(zeros_hbm.at[pl.ds(0, UWB)],
                              acc_sh.at[pl.ds(row0, UWB)], semz).wait()

    @pl.when(sid == NS - 1)
    def _zero_tail_wait():
        pltpu.make_async_copy(zeros_hbm,
                              acc_sh.at[pl.ds(row0, UWB + 16)], semz).wait()

    for b in range(NB - 1):
        _wait_idx(b, b)
        pltpu.async_copy(table_hbm.at[idx[b].at[0]], rows[b], semg[b])

    plsc.subcore_barrier()

    def _chunk(ch, b, i8):
        pltpu.make_async_copy(table_hbm.at[idx[i8].at[0]], rows[b],
                              semg[b]).wait()
        pltpu.async_copy(rows[b], acc_sh.at[idx[i8].at[1]], sems[b],
                         add=True)

        @pl.when(ch + NB < NCH)
        def _():
            i_n = (i8 + NB) % M
            pltpu.async_copy(eidx_hbm.at[pl.ds(0, 2), wid, ch + NB], idx[i_n], semi[i_n])

        @pl.when(ch + NB - 1 < NCH)
        def _():
            b2 = (b + NB - 1) % NB
            @pl.when(ch >= 1)
            def _():
                pltpu.make_async_copy(rows[b2], acc_sh.at[idx[0].at[1]],
                                      sems[b2]).wait()
            i2 = (i8 + NB - 1) % M
            pltpu.make_async_copy(eidx_hbm.at[pl.ds(0, 2), wid, 0], idx[i2],
                                  semi[i2]).wait()
            pltpu.async_copy(table_hbm.at[idx[i2].at[0]], rows[b2], semg[b2])

    @pl.loop(0, NCH // M)
    def _edges(g):
        ch0 = M * g
        for j in range(M):
            _chunk(ch0 + j, j % NB, j)

    for ch in range(NCH - NCH % M, NCH):
        _chunk(jnp.int32(ch), ch % NB, ch % M)

    for b in range(NB):
        pltpu.make_async_copy(rows[b], acc_sh.at[idx[0].at[1]],
                              sems[b]).wait()

    plsc.subcore_barrier()

    @pl.when(sid < NS - 1)
    def _wb():
        pltpu.sync_copy(acc_sh.at[pl.ds(row0, UWB)],
                        out_hbm.at[cid, pl.ds(row0, UWB)])

    @pl.when(sid == NS - 1)
    def _wb_tail():
        pltpu.sync_copy(acc_sh.at[pl.ds(row0, UWB + 16)],
                        out_hbm.at[cid, pl.ds(row0, UWB + 16)])


@functools.cache
def _make_seg_sum():
  return pl.kernel(
    _seg_sum_body,
    out_type=jax.ShapeDtypeStruct((NC, N, D), jnp.float32),
    mesh=plsc.VectorSubcoreMesh(core_axis_name="c", subcore_axis_name="s",
                                num_cores=NC, num_subcores=NS),
    scratch_types=(
        [pltpu.VMEM_SHARED((N, D), jnp.float32)]
        + [pltpu.VMEM((2, C), jnp.int32) for _ in range(M)]
        + [pltpu.VMEM((C, D), jnp.float32) for _ in range(NB)]
        + [pltpu.SemaphoreType.DMA] * (1 + M + NB + NB)
    ),
  )


BLK = 1000
GRID = N // BLK


def _layer1_body(p0, p1, x, w_rel, b_rel, w_root, out):
    agg = p0[...] + p1[...]
    h = (jnp.dot(agg, w_rel[...], preferred_element_type=jnp.float32)
         + jnp.dot(x[...], w_root[...], preferred_element_type=jnp.float32)
         + b_rel[...])
    out[...] = jnp.maximum(h, 0.0)


_layer1 = pl.pallas_call(
    _layer1_body,
    grid=(GRID,),
    in_specs=[
        pl.BlockSpec((BLK, D), lambda i: (i, 0)),
        pl.BlockSpec((BLK, D), lambda i: (i, 0)),
        pl.BlockSpec((BLK, D), lambda i: (i, 0)),
        pl.BlockSpec((D, D), lambda i: (0, 0)),
        pl.BlockSpec((1, D), lambda i: (0, 0)),
        pl.BlockSpec((D, D), lambda i: (0, 0)),
    ],
    out_specs=pl.BlockSpec((BLK, D), lambda i: (i, 0)),
    out_shape=jax.ShapeDtypeStruct((N, D), jnp.float32),
)


def _layer2_body(p0, p1, h1, w_rel, b_rel, w_root, w_count, b_count,
                 w_jet, b_jet, cnt_out, props_out, acc):
    i = pl.program_id(0)
    agg = p0[...] + p1[...]
    h = (jnp.dot(agg, w_rel[...], preferred_element_type=jnp.float32)
         + jnp.dot(h1[...], w_root[...], preferred_element_type=jnp.float32)
         + b_rel[...])
    h = jnp.maximum(h, 0.0)
    blk_sum = jnp.sum(h, axis=0, keepdims=True)

    @pl.when(i == 0)
    def _():
        acc[...] = jnp.zeros_like(acc)

    acc[...] += blk_sum

    @pl.when(i == GRID - 1)
    def _():
        pooled = acc[...] / float(N)
        cnt = jnp.dot(pooled, w_count[...],
                      preferred_element_type=jnp.float32) + b_count[...]
        cnt_out[...] = jnp.round(jnp.clip(cnt, 0.0, 20.0))
        props = jnp.dot(pooled, w_jet[...],
                        preferred_element_type=jnp.float32) + b_jet[...]
        props_out[...] = jnp.broadcast_to(props, (3, 5))


_layer2 = pl.pallas_call(
    _layer2_body,
    grid=(GRID,),
    in_specs=[
        pl.BlockSpec((BLK, D), lambda i: (i, 0)),
        pl.BlockSpec((BLK, D), lambda i: (i, 0)),
        pl.BlockSpec((BLK, D), lambda i: (i, 0)),
        pl.BlockSpec((D, D), lambda i: (0, 0)),
        pl.BlockSpec((1, D), lambda i: (0, 0)),
        pl.BlockSpec((D, D), lambda i: (0, 0)),
        pl.BlockSpec((D, 1), lambda i: (0, 0)),
        pl.BlockSpec((1, 1), lambda i: (0, 0)),
        pl.BlockSpec((D, 5), lambda i: (0, 0)),
        pl.BlockSpec((1, 5), lambda i: (0, 0)),
    ],
    out_specs=[
        pl.BlockSpec((1, 1), lambda i: (0, 0)),
        pl.BlockSpec((3, 5), lambda i: (0, 0)),
    ],
    out_shape=[
        jax.ShapeDtypeStruct((1, 1), jnp.float32),
        jax.ShapeDtypeStruct((3, 5), jnp.float32),
    ],
    scratch_shapes=[pltpu.VMEM((1, D), jnp.float32)],
)


@jax.jit
def kernel(x, edge_index, batch, W1_rel, b1_rel, W1_root, W2_rel, b2_rel,
           W2_root, W_count, b_count, W_jet, b_jet):
    eidx = edge_index.reshape(2, NW, NCH, C)
    zeros = jnp.zeros((UWB + 16, D), jnp.float32)

    _seg_sum = _make_seg_sum()
    parts1 = _seg_sum(eidx, x, zeros)
    h1 = _layer1(parts1[0], parts1[1], x, W1_rel, b1_rel.reshape(1, D),
                 W1_root)
    parts2 = _seg_sum(eidx, h1, zeros)
    jet_count, jet_properties = _layer2(
        parts2[0], parts2[1], h1, W2_rel, b2_rel.reshape(1, D), W2_root,
        W_count, b_count.reshape(1, 1), W_jet, b_jet.reshape(1, 5))
    return (jet_count, jet_properties)
